# baseline (speedup 1.0000x reference)
import jax
import jax.numpy as jnp
from jax.experimental import pallas as pl
from jax.experimental.pallas import tpu as pltpu

_NK = 8


def _body(t_ref, emb_ref, out_ref, tile_ref, sem_ref):
    del t_ref
    C, HWB = tile_ref.shape
    B = out_ref.shape[0] // C
    NK = out_ref.shape[1] // HWB
    tile_ref[...] = jax.lax.broadcast_in_dim(emb_ref[0], (C, HWB), (0, 1))
    copies = [
        pltpu.make_async_copy(
            tile_ref,
            out_ref.at[pl.ds(b * C, C), pl.ds(k * HWB, HWB)],
            sem_ref,
        )
        for b in range(B)
        for k in range(NK)
    ]
    for c in copies:
        c.start()
    for c in copies:
        c.wait()


def kernel(x, t, embeddings):
    B, _, H, W = x.shape
    C = embeddings.shape[1]
    HW = H * W
    HWB = HW // _NK
    t_arr = jnp.asarray(t, jnp.int32).reshape((1,))
    emb3 = embeddings.reshape(embeddings.shape[0], C, 1)
    grid_spec = pltpu.PrefetchScalarGridSpec(
        num_scalar_prefetch=1,
        grid=(1,),
        in_specs=[pl.BlockSpec((1, C, 1), lambda i, tr: (tr[0], 0, 0))],
        out_specs=pl.BlockSpec(memory_space=pl.ANY),
        scratch_shapes=[
            pltpu.VMEM((C, HWB), jnp.float32),
            pltpu.SemaphoreType.DMA,
        ],
    )
    out = pl.pallas_call(
        _body,
        grid_spec=grid_spec,
        out_shape=jax.ShapeDtypeStruct((B * C, HW), jnp.float32),
    )(t_arr, emb3)
    return out.reshape(B, C, H, W)


# speedup vs baseline: 1.4489x; 1.4489x over previous
import jax
import jax.numpy as jnp
from jax.experimental import pallas as pl
from jax.experimental.pallas import tpu as pltpu

_NK = 7


def _body(t_ref, emb_ref, out_ref, tile_ref, sem_ref):
    del t_ref
    _, C, HB, W = tile_ref.shape
    B = out_ref.shape[0]
    NK = out_ref.shape[2] // HB
    tile_ref[...] = jax.lax.broadcast_in_dim(emb_ref[0], (1, C, HB, W), (1, 2))
    copies = [
        pltpu.make_async_copy(
            tile_ref,
            out_ref.at[pl.ds(b, 1), :, pl.ds(k * HB, HB), :],
            sem_ref,
        )
        for b in range(B)
        for k in range(NK)
    ]
    for c in copies:
        c.start()
    for c in copies:
        c.wait()


def kernel(x, t, embeddings):
    B, _, H, W = x.shape
    C = embeddings.shape[1]
    HB = H // _NK
    t_arr = jnp.asarray(t, jnp.int32).reshape((1,))
    emb3 = embeddings.reshape(embeddings.shape[0], C, 1)
    grid_spec = pltpu.PrefetchScalarGridSpec(
        num_scalar_prefetch=1,
        grid=(1,),
        in_specs=[pl.BlockSpec((1, C, 1), lambda i, tr: (tr[0], 0, 0))],
        out_specs=pl.BlockSpec(memory_space=pl.ANY),
        scratch_shapes=[
            pltpu.VMEM((1, C, HB, W), jnp.float32),
            pltpu.SemaphoreType.DMA,
        ],
    )
    return pl.pallas_call(
        _body,
        grid_spec=grid_spec,
        out_shape=jax.ShapeDtypeStruct((B, C, H, W), jnp.float32),
    )(t_arr, emb3)


# speedup vs baseline: 1.4593x; 1.0072x over previous
import jax
import jax.numpy as jnp
from jax.experimental import pallas as pl
from jax.experimental.pallas import tpu as pltpu

_CB = 32


def _body(t_ref, emb_ref, out_ref):
    del t_ref
    out_ref[...] = jax.lax.broadcast_in_dim(emb_ref[0], out_ref.shape, (1, 2))


def kernel(x, t, embeddings):
    B, _, H, W = x.shape
    C = embeddings.shape[1]
    t_arr = jnp.asarray(t, jnp.int32).reshape((1,))
    emb3 = embeddings.reshape(embeddings.shape[0], C, 1)
    grid_spec = pltpu.PrefetchScalarGridSpec(
        num_scalar_prefetch=1,
        grid=(B, C // _CB),
        in_specs=[pl.BlockSpec((1, _CB, 1), lambda b, i, tr: (tr[0], i, 0))],
        out_specs=pl.BlockSpec((1, _CB, H, W), lambda b, i, tr: (b, i, 0, 0)),
    )
    return pl.pallas_call(
        _body,
        grid_spec=grid_spec,
        out_shape=jax.ShapeDtypeStruct((B, C, H, W), jnp.float32),
        compiler_params=pltpu.CompilerParams(
            dimension_semantics=("parallel", "parallel"),
        ),
    )(t_arr, emb3)
